# baseline (speedup 1.0000x reference)
import functools

import jax
import jax.numpy as jnp
from jax import lax
from jax.experimental import pallas as pl
from jax.experimental.pallas import tpu as pltpu
from jax.experimental.pallas import tpu_sc as plsc

B = 4096
F = 200
V = 100000
D = 64
EPS = 1e-5

NC = 2
NS = 16
NW = NC * NS

NSPLIT = 1
F_C = F // NSPLIT
QN = 800 // F_C
QB = B // QN
GROUPS = QB // 128
UPW = F_C * QN // NW


def _tbl_body(t_ref, gg_ref, gb_ref, vb_ref, o_ref):
    t = t_ref[...]
    mu = jnp.mean(t, axis=-1, keepdims=True)
    var = jnp.mean((t - mu) * (t - mu), axis=-1, keepdims=True)
    nt = (t - mu) * lax.rsqrt(var + EPS) * gg_ref[...] + (
        gb_ref[...] + vb_ref[...])
    o_ref[...] = nt


def _normalize_table(table, g_gamma, g_beta, v_beta):
    blk = 5000
    grid = V // blk
    return pl.pallas_call(
        _tbl_body,
        grid=(grid,),
        in_specs=[
            pl.BlockSpec((blk, D), lambda i: (i, 0)),
            pl.BlockSpec((1, D), lambda i: (0, 0)),
            pl.BlockSpec((1, D), lambda i: (0, 0)),
            pl.BlockSpec((1, D), lambda i: (0, 0)),
        ],
        out_specs=pl.BlockSpec((blk, D), lambda i: (i, 0)),
        out_shape=jax.ShapeDtypeStruct((V, D), jnp.float32),
    )(table, g_gamma.reshape(1, D), g_beta.reshape(1, D),
      v_beta.reshape(1, D))


def _aprep_body(x_ref, g_ref, w_ref, b_ref, vg_ref, a0_ref, a1_ref, a2_ref,
                gt_ref, wb_ref):
    w = w_ref[...]
    b = b_ref[...]
    vg = vg_ref[...]
    x = x_ref[...]
    mw = jnp.mean(w, axis=-1, keepdims=True)
    mb = jnp.mean(b, axis=-1, keepdims=True)
    mww = jnp.mean(w * w, axis=-1, keepdims=True)
    mwb = jnp.mean(w * b, axis=-1, keepdims=True)
    mbb = jnp.mean(b * b, axis=-1, keepdims=True)
    mu = x * mw + mb
    m2 = x * x * mww + 2.0 * x * mwb + mbb
    var = jnp.maximum(m2 - mu * mu, 0.0)
    rs = lax.rsqrt(var + EPS)
    shp3 = (F, B // 128, 128)
    a0_ref[...] = rs.reshape(shp3)
    a1_ref[...] = (rs * x).reshape(shp3)
    a2_ref[...] = (rs * mu).reshape(shp3)
    gt_ref[...] = g_ref[...].reshape(shp3)
    wb_ref[...] = jnp.concatenate(
        [(w * vg).reshape(F, 1, D), (b * vg).reshape(F, 1, D)], axis=1)


def _aprep(value_t, gene_t, weight, bias, v_gamma):
    shp = jax.ShapeDtypeStruct((F, B // 128, 128), jnp.float32)
    return pl.pallas_call(
        _aprep_body,
        out_shape=[shp, shp, shp,
                   jax.ShapeDtypeStruct((F, B // 128, 128), jnp.int32),
                   jax.ShapeDtypeStruct((F, 2, D), jnp.float32)],
    )(value_t, gene_t, weight, bias, v_gamma.reshape(1, D))


def _make_sc_body(f_base):
    def _sc_body(ntable, gene_t, a0h, a1h, a2h, wb, vgh, out_hbm,
                 idx_v, a0_v, a1_v, a2_v, wb_v, vg_v,
                 gbuf0, gbuf1, gbuf2, sbuf0, sbuf1, ridx_v,
                 sem_g0, sem_g1, sem_g2, sem_s0, sem_s1, sem_h):
        gbufs = (gbuf0, gbuf1, gbuf2)
        sbufs = (sbuf0, sbuf1)
        sem_g = (sem_g0, sem_g1, sem_g2)
        sem_s = (sem_s0, sem_s1)
        cid = lax.axis_index("c")
        sid = lax.axis_index("s")
        wid = sid * NC + cid
        iota = lax.iota(jnp.int32, 16)
        iota_f = iota * F_C

        pltpu.sync_copy(vgh, vg_v)
        vgc = [vg_v[pl.ds(c * 16, 16)] for c in range(4)]

        def _head_views(p):
            return (wb_v.at[pl.ds(p * 2, 2)], idx_v.at[pl.ds(p * GROUPS, 8)],
                    a0_v.at[pl.ds(p * GROUPS, 8)],
                    a1_v.at[pl.ds(p * GROUPS, 8)],
                    a2_v.at[pl.ds(p * GROUPS, 8)])

        def _head_srcs(k):
            u = wid * UPW + k
            fl = u // QN
            q = u - fl * QN
            f = f_base + fl
            qs = pl.ds(q * GROUPS, GROUPS)
            return (wb.at[f], gene_t.at[f, qs], a0h.at[f, qs],
                    a1h.at[f, qs], a2h.at[f, qs])

        def _issue_heads(k, p):
            for s, d in zip(_head_srcs(k), _head_views(p)):
                pltpu.async_copy(s, d, sem_h)

        _issue_heads(0, 0)

        def unit_body(k, carry):
            u = wid * UPW + k
            fl = u // QN
            q = u - fl * QN
            qb = q * QB
            p = k - (k // 2) * 2
            p8 = p * GROUPS

            for s, d in zip(_head_srcs(k), _head_views(p)):
                pltpu.make_async_copy(s, d, sem_h).wait()
            kn = jnp.minimum(k + 1, UPW - 1)
            _issue_heads(kn, 1 - p)

            w1c = [wb_v[p * 2, pl.ds(c * 16, 16)] for c in range(4)]
            b1c = [wb_v[p * 2 + 1, pl.ds(c * 16, 16)] for c in range(4)]

            base = qb * F_C + fl
            for j in range(GROUPS):
                for c in range(8):
                    ridx_v[j, c * 16:(c + 1) * 16] = (
                        iota_f + (base + (j * 128 + c * 16) * F_C))

            lane_ids = [jnp.full((16, 1), r2, jnp.int32) for r2 in range(16)]
            _gdn = lax.GatherDimensionNumbers(
                offset_dims=(), collapsed_slice_dims=(0,),
                start_index_map=(0,))

            def _bcast_lane(vec, r2):
                return lax.gather(vec, lane_ids[r2], _gdn, (1,),
                                  mode=lax.GatherScatterMode.PROMISE_IN_BOUNDS)

            def compute_group(j, gbuf, sbuf):
                @plsc.parallel_loop(0, 8, unroll=2)
                def chunk_body(rc):
                    a0vec = a0_v[p8 + j, pl.ds(rc * 16, 16)]
                    a1vec = a1_v[p8 + j, pl.ds(rc * 16, 16)]
                    a2vec = a2_v[p8 + j, pl.ds(rc * 16, 16)]
                    r0 = rc * 16
                    for r2 in range(16):
                        a0 = _bcast_lane(a0vec, r2)
                        a1 = _bcast_lane(a1vec, r2)
                        a2 = _bcast_lane(a2vec, r2)
                        r = r0 + r2
                        for c in range(4):
                            sl = pl.ds(c * 16, 16)
                            g = gbuf[r, sl]
                            sbuf[r, sl] = (g + w1c[c] * a1 + b1c[c] * a0
                                           - vgc[c] * a2)

            def gather(j):
                return pltpu.async_copy(
                    ntable.at[idx_v.at[p8 + j]], gbufs[j % 3], sem_g[j % 3])

            def scatter(j):
                return pltpu.async_copy(
                    sbufs[j % 2], out_hbm.at[ridx_v.at[j]], sem_s[j % 2])

            dg = {0: gather(0), 1: gather(1), 2: gather(2)}
            ds = {}
            for j in range(GROUPS):
                dg[j].wait()
                if j >= 2:
                    ds[j - 2].wait()
                compute_group(j, gbufs[j % 3], sbufs[j % 2])
                ds[j] = scatter(j)
                if j + 3 < GROUPS:
                    dg[j + 3] = gather(j + 3)
            ds[GROUPS - 2].wait()
            ds[GROUPS - 1].wait()
            return carry

        lax.fori_loop(0, UPW, unit_body, 0)
        for s, d in zip(_head_srcs(UPW - 1),
                        _head_views(1 - (UPW - 1) % 2)):
            pltpu.make_async_copy(s, d, sem_h).wait()

    return _sc_body


@functools.partial(jax.jit)
def _encode(gene, value, table, g_gamma, g_beta, weight, bias, v_gamma,
            v_beta):
    ntable = _normalize_table(table, g_gamma, g_beta, v_beta)
    a0, a1, a2, gene_t, wb = _aprep(
        value.T, gene.astype(jnp.int32).T, weight, bias, v_gamma)

    scratch_types = [
        pltpu.VMEM((2 * GROUPS, 128), jnp.int32),
        pltpu.VMEM((2 * GROUPS, 128), jnp.float32),
        pltpu.VMEM((2 * GROUPS, 128), jnp.float32),
        pltpu.VMEM((2 * GROUPS, 128), jnp.float32),
        pltpu.VMEM((4, D), jnp.float32),
        pltpu.VMEM((D,), jnp.float32),
        pltpu.VMEM((128, D), jnp.float32),
        pltpu.VMEM((128, D), jnp.float32),
        pltpu.VMEM((128, D), jnp.float32),
        pltpu.VMEM((128, D), jnp.float32),
        pltpu.VMEM((128, D), jnp.float32),
        pltpu.VMEM((GROUPS, 128), jnp.int32),
        pltpu.SemaphoreType.DMA,
        pltpu.SemaphoreType.DMA,
        pltpu.SemaphoreType.DMA,
        pltpu.SemaphoreType.DMA,
        pltpu.SemaphoreType.DMA,
        pltpu.SemaphoreType.DMA,
    ]
    outs = []
    for s in range(NSPLIT):
        sc = pl.kernel(
            _make_sc_body(s * F_C),
            out_type=jax.ShapeDtypeStruct((B * F_C, D), jnp.float32),
            mesh=plsc.VectorSubcoreMesh(core_axis_name="c",
                                        subcore_axis_name="s"),
            compiler_params=pltpu.CompilerParams(use_tc_tiling_on_sc=False),
            scratch_types=scratch_types,
        )
        outs.append(
            sc(ntable, gene_t, a0, a1, a2, wb, v_gamma).reshape(B, F_C, D))
    return jnp.concatenate(outs, axis=1)


def kernel(gene, value, table, g_gamma, g_beta, weight, bias, v_gamma,
           v_beta):
    return _encode(gene, value, table, g_gamma, g_beta, weight, bias,
                   v_gamma, v_beta)


# speedup vs baseline: 1.0024x; 1.0024x over previous
import functools

import jax
import jax.numpy as jnp
from jax import lax
from jax.experimental import pallas as pl
from jax.experimental.pallas import tpu as pltpu
from jax.experimental.pallas import tpu_sc as plsc

B = 4096
F = 200
V = 100000
D = 64
EPS = 1e-5

NC = 2
NS = 16
NW = NC * NS

NSPLIT = 1
F_C = F // NSPLIT
QN = 800 // F_C
QB = B // QN
GROUPS = QB // 128
UPW = F_C * QN // NW


def _tbl_body(t_ref, gg_ref, gb_ref, vb_ref, o_ref):
    t = t_ref[...]
    mu = jnp.mean(t, axis=-1, keepdims=True)
    var = jnp.mean((t - mu) * (t - mu), axis=-1, keepdims=True)
    nt = (t - mu) * lax.rsqrt(var + EPS) * gg_ref[...] + (
        gb_ref[...] + vb_ref[...])
    o_ref[...] = nt


def _normalize_table(table, g_gamma, g_beta, v_beta):
    blk = 5000
    grid = V // blk
    return pl.pallas_call(
        _tbl_body,
        grid=(grid,),
        in_specs=[
            pl.BlockSpec((blk, D), lambda i: (i, 0)),
            pl.BlockSpec((1, D), lambda i: (0, 0)),
            pl.BlockSpec((1, D), lambda i: (0, 0)),
            pl.BlockSpec((1, D), lambda i: (0, 0)),
        ],
        out_specs=pl.BlockSpec((blk, D), lambda i: (i, 0)),
        out_shape=jax.ShapeDtypeStruct((V, D), jnp.float32),
    )(table, g_gamma.reshape(1, D), g_beta.reshape(1, D),
      v_beta.reshape(1, D))


def _aprep_body(x_ref, g_ref, w_ref, b_ref, vg_ref, a0_ref, a1_ref, a2_ref,
                gt_ref, wb_ref):
    w = w_ref[...]
    b = b_ref[...]
    vg = vg_ref[...]
    x = x_ref[...]
    mw = jnp.mean(w, axis=-1, keepdims=True)
    mb = jnp.mean(b, axis=-1, keepdims=True)
    mww = jnp.mean(w * w, axis=-1, keepdims=True)
    mwb = jnp.mean(w * b, axis=-1, keepdims=True)
    mbb = jnp.mean(b * b, axis=-1, keepdims=True)
    mu = x * mw + mb
    m2 = x * x * mww + 2.0 * x * mwb + mbb
    var = jnp.maximum(m2 - mu * mu, 0.0)
    rs = lax.rsqrt(var + EPS)
    shp3 = (F, B // 128, 128)
    a0_ref[...] = rs.reshape(shp3)
    a1_ref[...] = (rs * x).reshape(shp3)
    a2_ref[...] = (rs * mu).reshape(shp3)
    gt_ref[...] = g_ref[...].reshape(shp3)
    wb_ref[...] = jnp.concatenate(
        [(w * vg).reshape(F, 1, D), (b * vg).reshape(F, 1, D)], axis=1)


def _aprep(value_t, gene_t, weight, bias, v_gamma):
    shp = jax.ShapeDtypeStruct((F, B // 128, 128), jnp.float32)
    return pl.pallas_call(
        _aprep_body,
        out_shape=[shp, shp, shp,
                   jax.ShapeDtypeStruct((F, B // 128, 128), jnp.int32),
                   jax.ShapeDtypeStruct((F, 2, D), jnp.float32)],
    )(value_t, gene_t, weight, bias, v_gamma.reshape(1, D))


def _make_sc_body(f_base):
    def _sc_body(ntable, gene_t, a0h, a1h, a2h, wb, vgh, out_hbm,
                 idx_v, a0_v, a1_v, a2_v, wb_v, vg_v,
                 gbuf0, gbuf1, sbuf0, sbuf1, ridx_v,
                 sem_g0, sem_g1, sem_s0, sem_s1, sem_h):
        gbufs = (gbuf0, gbuf1)
        sbufs = (sbuf0, sbuf1)
        sem_g = (sem_g0, sem_g1)
        sem_s = (sem_s0, sem_s1)
        cid = lax.axis_index("c")
        sid = lax.axis_index("s")
        wid = sid * NC + cid
        iota = lax.iota(jnp.int32, 16)
        iota_f = iota * F_C

        pltpu.sync_copy(vgh, vg_v)
        vgc = [vg_v[pl.ds(c * 16, 16)] for c in range(4)]

        def _head_views(p):
            return (wb_v.at[pl.ds(p * 2, 2)], idx_v.at[pl.ds(p * GROUPS, 8)],
                    a0_v.at[pl.ds(p * GROUPS, 8)],
                    a1_v.at[pl.ds(p * GROUPS, 8)],
                    a2_v.at[pl.ds(p * GROUPS, 8)])

        def _head_srcs(k):
            u = wid * UPW + k
            fl = u // QN
            q = u - fl * QN
            f = f_base + fl
            qs = pl.ds(q * GROUPS, GROUPS)
            return (wb.at[f], gene_t.at[f, qs], a0h.at[f, qs],
                    a1h.at[f, qs], a2h.at[f, qs])

        def _issue_heads(k, p):
            for s, d in zip(_head_srcs(k), _head_views(p)):
                pltpu.async_copy(s, d, sem_h)

        _issue_heads(0, 0)

        def unit_body(k, carry):
            u = wid * UPW + k
            fl = u // QN
            q = u - fl * QN
            qb = q * QB
            p = k - (k // 2) * 2
            p8 = p * GROUPS

            for s, d in zip(_head_srcs(k), _head_views(p)):
                pltpu.make_async_copy(s, d, sem_h).wait()
            kn = jnp.minimum(k + 1, UPW - 1)
            _issue_heads(kn, 1 - p)

            w1c = [wb_v[p * 2, pl.ds(c * 16, 16)] for c in range(4)]
            b1c = [wb_v[p * 2 + 1, pl.ds(c * 16, 16)] for c in range(4)]

            base = qb * F_C + fl
            for j in range(GROUPS):
                for c in range(8):
                    ridx_v[j, c * 16:(c + 1) * 16] = (
                        iota_f + (base + (j * 128 + c * 16) * F_C))

            lane_ids = [jnp.full((16, 1), r2, jnp.int32) for r2 in range(16)]
            _gdn = lax.GatherDimensionNumbers(
                offset_dims=(), collapsed_slice_dims=(0,),
                start_index_map=(0,))

            def _bcast_lane(vec, r2):
                return lax.gather(vec, lane_ids[r2], _gdn, (1,),
                                  mode=lax.GatherScatterMode.PROMISE_IN_BOUNDS)

            def compute_group(j, gbuf, sbuf):
                @plsc.parallel_loop(0, 8, unroll=2)
                def chunk_body(rc):
                    a0vec = a0_v[p8 + j, pl.ds(rc * 16, 16)]
                    a1vec = a1_v[p8 + j, pl.ds(rc * 16, 16)]
                    a2vec = a2_v[p8 + j, pl.ds(rc * 16, 16)]
                    r0 = rc * 16
                    for r2 in range(16):
                        a0 = _bcast_lane(a0vec, r2)
                        a1 = _bcast_lane(a1vec, r2)
                        a2 = _bcast_lane(a2vec, r2)
                        r = r0 + r2
                        for c in range(4):
                            sl = pl.ds(c * 16, 16)
                            g = gbuf[r, sl]
                            sbuf[r, sl] = (g + w1c[c] * a1 + b1c[c] * a0
                                           - vgc[c] * a2)

            def gather(j):
                return pltpu.async_copy(
                    ntable.at[idx_v.at[p8 + j]], gbufs[j % 2], sem_g[j % 2])

            def scatter(j):
                return pltpu.async_copy(
                    sbufs[j % 2], out_hbm.at[ridx_v.at[j]], sem_s[j % 2])

            dg = {0: gather(0), 1: gather(1)}
            ds = {}
            for j in range(GROUPS):
                dg[j].wait()
                if j >= 2:
                    ds[j - 2].wait()
                compute_group(j, gbufs[j % 2], sbufs[j % 2])
                ds[j] = scatter(j)
                if j + 2 < GROUPS:
                    dg[j + 2] = gather(j + 2)
            ds[GROUPS - 2].wait()
            ds[GROUPS - 1].wait()
            return carry

        lax.fori_loop(0, UPW, unit_body, 0)
        for s, d in zip(_head_srcs(UPW - 1),
                        _head_views(1 - (UPW - 1) % 2)):
            pltpu.make_async_copy(s, d, sem_h).wait()

    return _sc_body


@functools.partial(jax.jit)
def _encode(gene, value, table, g_gamma, g_beta, weight, bias, v_gamma,
            v_beta):
    ntable = _normalize_table(table, g_gamma, g_beta, v_beta)
    a0, a1, a2, gene_t, wb = _aprep(
        value.T, gene.astype(jnp.int32).T, weight, bias, v_gamma)

    scratch_types = [
        pltpu.VMEM((2 * GROUPS, 128), jnp.int32),
        pltpu.VMEM((2 * GROUPS, 128), jnp.float32),
        pltpu.VMEM((2 * GROUPS, 128), jnp.float32),
        pltpu.VMEM((2 * GROUPS, 128), jnp.float32),
        pltpu.VMEM((4, D), jnp.float32),
        pltpu.VMEM((D,), jnp.float32),
        pltpu.VMEM((128, D), jnp.float32),
        pltpu.VMEM((128, D), jnp.float32),
        pltpu.VMEM((128, D), jnp.float32),
        pltpu.VMEM((128, D), jnp.float32),
        pltpu.VMEM((GROUPS, 128), jnp.int32),
        pltpu.SemaphoreType.DMA,
        pltpu.SemaphoreType.DMA,
        pltpu.SemaphoreType.DMA,
        pltpu.SemaphoreType.DMA,
        pltpu.SemaphoreType.DMA,
    ]
    outs = []
    for s in range(NSPLIT):
        sc = pl.kernel(
            _make_sc_body(s * F_C),
            out_type=jax.ShapeDtypeStruct((B * F_C, D), jnp.float32),
            mesh=plsc.VectorSubcoreMesh(core_axis_name="c",
                                        subcore_axis_name="s"),
            compiler_params=pltpu.CompilerParams(use_tc_tiling_on_sc=False),
            scratch_types=scratch_types,
        )
        outs.append(
            sc(ntable, gene_t, a0, a1, a2, wb, v_gamma).reshape(B, F_C, D))
    return jnp.concatenate(outs, axis=1)


def kernel(gene, value, table, g_gamma, g_beta, weight, bias, v_gamma,
           v_beta):
    return _encode(gene, value, table, g_gamma, g_beta, weight, bias,
                   v_gamma, v_beta)
